# Initial kernel scaffold; baseline (speedup 1.0000x reference)
import jax
import jax.numpy as jnp
from jax.experimental import pallas as pl


def kernel(x, cluster):
    raise NotImplementedError("write your pallas kernel here")


# speedup vs baseline: 1.0544x; 1.0544x over previous
import jax
import jax.numpy as jnp
from jax.experimental import pallas as pl
from jax.experimental.pallas import tpu as pltpu

_N = 65536
_NDIM = 32
_K = 512
_TILE = 2048
_GRID = _N // _TILE


def _assign_kernel(x_ref, c_ref, idx_ref, loss_ref):
    x = x_ref[...]
    c = c_ref[...]
    x2 = jnp.sum(x * x, axis=1, keepdims=True)
    c2 = jnp.sum(c * c, axis=1)[None, :]
    s = jax.lax.dot_general(
        x, c, (((1,), (1,)), ((), ())),
        preferred_element_type=jnp.float32)
    d = x2 + c2 - 2.0 * s
    idx_ref[0, 0, :] = jnp.argmin(d, axis=1).astype(jnp.int32)
    partial = jnp.sum(jnp.min(d, axis=1))

    @pl.when(pl.program_id(0) == 0)
    def _():
        loss_ref[0, 0] = 0.0

    loss_ref[0, 0] += partial


def kernel(x, cluster):
    idx2d, loss_sum = pl.pallas_call(
        _assign_kernel,
        grid=(_GRID,),
        in_specs=[
            pl.BlockSpec((_TILE, _NDIM), lambda i: (i, 0)),
            pl.BlockSpec((_K, _NDIM), lambda i: (0, 0)),
        ],
        out_specs=[
            pl.BlockSpec((1, 1, _TILE), lambda i: (i, 0, 0)),
            pl.BlockSpec((1, 1), lambda i: (0, 0), memory_space=pltpu.SMEM),
        ],
        out_shape=[
            jax.ShapeDtypeStruct((_GRID, 1, _TILE), jnp.int32),
            jax.ShapeDtypeStruct((1, 1), jnp.float32),
        ],
    )(x, cluster)
    idx = idx2d.reshape(_N)
    loss = loss_sum[0, 0] / jnp.float32(_N)
    return (idx, loss)


# speedup vs baseline: 1.8419x; 1.7469x over previous
import jax
import jax.numpy as jnp
from jax.experimental import pallas as pl
from jax.experimental.pallas import tpu as pltpu

_N = 65536
_NDIM = 32
_K = 512
_TILE = 2048
_GRID = _N // _TILE


def _assign_kernel(x_ref, c_ref, idx_ref, loss_ref):
    x = x_ref[...]
    c = c_ref[...]
    xsq = x * x
    x2 = jax.lax.dot_general(
        jnp.ones((1, _NDIM), jnp.float32), xsq, (((1,), (1,)), ((), ())),
        preferred_element_type=jnp.float32)
    c2 = jnp.sum(c * c, axis=1, keepdims=True)
    s = jax.lax.dot_general(
        c, x, (((1,), (1,)), ((), ())),
        preferred_element_type=jnp.float32)
    d = (x2 + c2) - 2.0 * s
    idx_ref[0, 0, :] = jnp.argmin(d, axis=0).astype(jnp.int32)
    partial = jnp.sum(jnp.min(d, axis=0))

    @pl.when(pl.program_id(0) == 0)
    def _():
        loss_ref[0, 0] = 0.0

    loss_ref[0, 0] += partial


def kernel(x, cluster):
    idx2d, loss_sum = pl.pallas_call(
        _assign_kernel,
        grid=(_GRID,),
        in_specs=[
            pl.BlockSpec((_TILE, _NDIM), lambda i: (i, 0)),
            pl.BlockSpec((_K, _NDIM), lambda i: (0, 0)),
        ],
        out_specs=[
            pl.BlockSpec((1, 1, _TILE), lambda i: (i, 0, 0)),
            pl.BlockSpec((1, 1), lambda i: (0, 0), memory_space=pltpu.SMEM),
        ],
        out_shape=[
            jax.ShapeDtypeStruct((_GRID, 1, _TILE), jnp.int32),
            jax.ShapeDtypeStruct((1, 1), jnp.float32),
        ],
    )(x, cluster)
    idx = idx2d.reshape(_N)
    loss = loss_sum[0, 0] / jnp.float32(_N)
    return (idx, loss)


# speedup vs baseline: 2.2701x; 1.2325x over previous
import jax
import jax.numpy as jnp
from jax.experimental import pallas as pl
from jax.experimental.pallas import tpu as pltpu

_N = 65536
_NDIM = 32
_K = 512
_TILE = 8192
_GRID = _N // _TILE


def _assign_kernel(x_ref, c_ref, idx_ref, loss_ref):
    x = x_ref[...]
    c = c_ref[...]
    xsq = x * x
    x2 = jax.lax.dot_general(
        jnp.ones((1, _NDIM), jnp.float32), xsq, (((1,), (1,)), ((), ())),
        preferred_element_type=jnp.float32)
    ch2 = 0.5 * jnp.sum(c * c, axis=1, keepdims=True)
    s = jax.lax.dot_general(
        c, x, (((1,), (1,)), ((), ())),
        preferred_element_type=jnp.float32)
    d = ch2 - s
    idx_ref[0, 0, :] = jnp.argmin(d, axis=0).astype(jnp.int32)
    partial = jnp.sum(x2) + 2.0 * jnp.sum(jnp.min(d, axis=0))

    @pl.when(pl.program_id(0) == 0)
    def _():
        loss_ref[0, 0] = 0.0

    loss_ref[0, 0] += partial


def kernel(x, cluster):
    idx2d, loss_sum = pl.pallas_call(
        _assign_kernel,
        grid=(_GRID,),
        in_specs=[
            pl.BlockSpec((_TILE, _NDIM), lambda i: (i, 0)),
            pl.BlockSpec((_K, _NDIM), lambda i: (0, 0)),
        ],
        out_specs=[
            pl.BlockSpec((1, 1, _TILE), lambda i: (i, 0, 0)),
            pl.BlockSpec((1, 1), lambda i: (0, 0), memory_space=pltpu.SMEM),
        ],
        out_shape=[
            jax.ShapeDtypeStruct((_GRID, 1, _TILE), jnp.int32),
            jax.ShapeDtypeStruct((1, 1), jnp.float32),
        ],
    )(x, cluster)
    idx = idx2d.reshape(_N)
    loss = loss_sum[0, 0] / jnp.float32(_N)
    return (idx, loss)
